# baseline (speedup 1.0000x reference)
import functools

import jax
import jax.numpy as jnp
from jax import lax
from jax.experimental import pallas as pl
from jax.experimental.pallas import tpu as pltpu
from jax.experimental.pallas import tpu_sc as plsc

N = 10000
D = 128
E = 320000
NC, NS = 2, 16
NW = NC * NS
C = 80
PER_W = E // NW
CPW = PER_W // C
NPH = 5
M = CPW // NPH
ZC = 40
NB = 4
GL = 2
SL = 1
NP = 10240
RPT = NP // NS
ZR = 128
DW = 128
BN = 2000
RSCALE = 1.0 / (1.0 + 1e-05) ** 0.5


def _sc_agg_body(x_hbm, src_hbm, dst_hbm, out_hbm,
                 acc, sidx, didx, rows, *sems):
    gsems, ssems = sems[:NB], sems[NB:]
    cc = lax.axis_index("c")
    ss = lax.axis_index("s")
    wid = cc * NS + ss

    def zrow(r, carry):
        for q in range(D // 16):
            rows[0, r, pl.ds(q * 16, 16)] = jnp.zeros((16,), jnp.float32)
        return carry
    lax.fori_loop(0, ZC, zrow, None)
    for k in range(RPT // ZC):
        pltpu.sync_copy(rows.at[0, pl.ds(0, ZC)],
                        acc.at[pl.ds(ss * RPT + k * ZC, ZC)])
    plsc.subcore_barrier()

    def fire_g(j, b):
        pltpu.async_copy(x_hbm.at[sidx.at[j]], rows.at[b], gsems[b])

    def wait_g(b):
        pltpu.make_async_copy(x_hbm.at[pl.ds(0, C)], rows.at[b],
                              gsems[b]).wait()

    def fire_s(j, b):
        pltpu.async_copy(rows.at[b], acc.at[didx.at[j]], ssems[b], add=True)

    def wait_s(b):
        pltpu.make_async_copy(rows.at[b], acc.at[pl.ds(0, C)],
                              ssems[b]).wait()

    for ph in range(NPH):
        pltpu.sync_copy(src_hbm.at[wid, ph], sidx)
        pltpu.sync_copy(dst_hbm.at[wid, ph], didx)
        for j in range(GL):
            fire_g(j, j % NB)
        for j in range(GL, NB):
            fire_g(j, j % NB)
            wait_g((j - GL) % NB)
            fire_s(j - GL, (j - GL) % NB)

        def steady(t, carry):
            for b in range(NB):
                j = NB + t * NB + b
                wait_s(b)
                fire_g(j, b)
                bp = (b - GL) % NB
                wait_g(bp)
                fire_s(j - GL, bp)
            return carry
        nsteady = (M - NB) // NB
        lax.fori_loop(0, nsteady, steady, None)
        for j in range(NB + nsteady * NB, M):
            b = j % NB
            wait_s(b)
            fire_g(j, b)
            bp = (j - GL) % NB
            wait_g(bp)
            fire_s(j - GL, bp)
        for jj in range(M - GL, M):
            bp = jj % NB
            wait_g(bp)
            fire_s(jj, bp)
        for b in range(NB):
            wait_s(b)

    plsc.subcore_barrier()
    pltpu.sync_copy(acc.at[pl.ds(ss * RPT, RPT)],
                    out_hbm.at[cc, pl.ds(ss * RPT, RPT)])


@functools.lru_cache(maxsize=None)
def _make_sc_agg():
    mesh = plsc.VectorSubcoreMesh(core_axis_name="c", subcore_axis_name="s",
                                  num_cores=NC, num_subcores=NS)
    return pl.kernel(
        _sc_agg_body,
        out_type=(jax.ShapeDtypeStruct((NC, NP, D), jnp.float32),),
        mesh=mesh,
        scratch_types=(
            pltpu.VMEM_SHARED((NP, D), jnp.float32),
            pltpu.VMEM((M, C), jnp.int32),
            pltpu.VMEM((M, C), jnp.int32),
            pltpu.VMEM((NB, C, D), jnp.float32),
        ) + (pltpu.SemaphoreType.DMA,) * (2 * NB),
    )


def _sc_deg_body(dst_hbm, dout_hbm, dacc, didx, ones, ssem):
    cc = lax.axis_index("c")
    ss = lax.axis_index("s")
    wid = cc * NS + ss

    def zrow(r, carry):
        for q in range(DW // 16):
            ones[r, pl.ds(q * 16, 16)] = jnp.zeros((16,), jnp.float32)
        return carry
    lax.fori_loop(0, C, zrow, None)
    for k in range(RPT // ZC):
        pltpu.sync_copy(ones.at[pl.ds(0, ZC)],
                        dacc.at[pl.ds(ss * RPT + k * ZC, ZC)])

    def onesrow(r, carry):
        for q in range(DW // 16):
            ones[r, pl.ds(q * 16, 16)] = jnp.ones((16,), jnp.float32)
        return carry
    lax.fori_loop(0, C, onesrow, None)
    plsc.subcore_barrier()

    def wait_s():
        pltpu.make_async_copy(ones, dacc.at[pl.ds(0, C)], ssem).wait()

    for ph in range(NPH):
        pltpu.sync_copy(dst_hbm.at[wid, ph], didx)
        for j in range(SL + GL):
            pltpu.async_copy(ones, dacc.at[didx.at[j]], ssem, add=True)

        def chunk(j, carry):
            pltpu.async_copy(ones, dacc.at[didx.at[j]], ssem, add=True)
            wait_s()
            return carry
        lax.fori_loop(SL + GL, M, chunk, None)
        for _ in range(SL + GL):
            wait_s()

    plsc.subcore_barrier()
    pltpu.sync_copy(dacc.at[pl.ds(ss * RPT, RPT)],
                    dout_hbm.at[cc, pl.ds(ss * RPT, RPT)])


@functools.lru_cache(maxsize=None)
def _make_sc_deg():
    mesh = plsc.VectorSubcoreMesh(core_axis_name="c", subcore_axis_name="s",
                                  num_cores=NC, num_subcores=NS)
    return pl.kernel(
        _sc_deg_body,
        out_type=(jax.ShapeDtypeStruct((NC, NP, DW), jnp.float32),),
        mesh=mesh,
        scratch_types=(
            pltpu.VMEM_SHARED((NP, DW), jnp.float32),
            pltpu.VMEM((M, C), jnp.int32),
            pltpu.VMEM((C, DW), jnp.float32),
            pltpu.SemaphoreType.DMA,
        ),
    )


def _tc_layer_body(act, p_ref, d_ref, x_ref, wl_ref, wr_ref, b_ref,
                   g_ref, be_ref, o_ref):
    deg = jnp.maximum(d_ref[0, :, 0:1] + d_ref[1, :, 0:1], 1.0)
    agg = (p_ref[0] + p_ref[1]) / deg
    y = (jnp.dot(agg, wl_ref[...], preferred_element_type=jnp.float32)
         + jnp.dot(x_ref[...], wr_ref[...], preferred_element_type=jnp.float32)
         + b_ref[...])
    if act == "lrelu":
        y = jnp.where(y >= 0, y, 0.01 * y)
    else:
        y = jnp.maximum(y, 0.0)
    o_ref[...] = y * (RSCALE * g_ref[...]) + be_ref[...]


def _make_tc_layer(act):
    grid = (N // BN,)
    in_specs = [
        pl.BlockSpec((NC, BN, D), lambda i: (0, i, 0)),
        pl.BlockSpec((NC, BN, DW), lambda i: (0, i, 0)),
        pl.BlockSpec((BN, D), lambda i: (i, 0)),
        pl.BlockSpec((D, D), lambda i: (0, 0)),
        pl.BlockSpec((D, D), lambda i: (0, 0)),
        pl.BlockSpec((1, D), lambda i: (0, 0)),
        pl.BlockSpec((1, D), lambda i: (0, 0)),
        pl.BlockSpec((1, D), lambda i: (0, 0)),
    ]
    return pl.pallas_call(
        functools.partial(_tc_layer_body, act),
        grid=grid,
        in_specs=in_specs,
        out_specs=pl.BlockSpec((BN, D), lambda i: (i, 0)),
        out_shape=jax.ShapeDtypeStruct((N, D), jnp.float32),
    )


_tc_layer_lrelu = _make_tc_layer("lrelu")
_tc_layer_relu = _make_tc_layer("relu")


def _tc_final_body(p_ref, d_ref, x_ref, wl_ref, wr_ref, b_ref,
                   wm0_ref, gm0_ref, bm0_ref, wm1_ref, gm1_ref, bm1_ref,
                   wm2_ref, bm2_ref, o_ref):
    deg = jnp.maximum(d_ref[0, :, 0:1] + d_ref[1, :, 0:1], 1.0)
    agg = (p_ref[0] + p_ref[1]) / deg
    y = (jnp.dot(agg, wl_ref[...], preferred_element_type=jnp.float32)
         + jnp.dot(x_ref[...], wr_ref[...], preferred_element_type=jnp.float32)
         + b_ref[...])
    h = jnp.dot(y, wm0_ref[...], preferred_element_type=jnp.float32)
    h = jnp.maximum(h * (RSCALE * gm0_ref[...]) + bm0_ref[...], 0.0)
    h = jnp.dot(h, wm1_ref[...], preferred_element_type=jnp.float32)
    h = jnp.maximum(h * (RSCALE * gm1_ref[...]) + bm1_ref[...], 0.0)
    o_ref[...] = (jnp.dot(h, wm2_ref[...], preferred_element_type=jnp.float32)
                  + bm2_ref[...])


_NCLS = 21

_tc_final = pl.pallas_call(
    _tc_final_body,
    grid=(N // BN,),
    in_specs=[
        pl.BlockSpec((NC, BN, D), lambda i: (0, i, 0)),
        pl.BlockSpec((NC, BN, DW), lambda i: (0, i, 0)),
        pl.BlockSpec((BN, D), lambda i: (i, 0)),
        pl.BlockSpec((D, D), lambda i: (0, 0)),
        pl.BlockSpec((D, D), lambda i: (0, 0)),
        pl.BlockSpec((1, D), lambda i: (0, 0)),
        pl.BlockSpec((D, D), lambda i: (0, 0)),
        pl.BlockSpec((1, D), lambda i: (0, 0)),
        pl.BlockSpec((1, D), lambda i: (0, 0)),
        pl.BlockSpec((D, D), lambda i: (0, 0)),
        pl.BlockSpec((1, D), lambda i: (0, 0)),
        pl.BlockSpec((1, D), lambda i: (0, 0)),
        pl.BlockSpec((D, _NCLS), lambda i: (0, 0)),
        pl.BlockSpec((1, _NCLS), lambda i: (0, 0)),
    ],
    out_specs=pl.BlockSpec((BN, _NCLS), lambda i: (i, 0)),
    out_shape=jax.ShapeDtypeStruct((N, _NCLS), jnp.float32),
)


def kernel(x1, edge_index, Wl1, Wr1, b1, Wl2, Wr2, b2, Wl3, Wr3, b3,
           Wl4, Wr4, b4, g1, be1, g2, be2, g3, be3,
           Wm0, gm0, bm0, Wm1, gm1, bm1, Wm2, bm2):
    src3 = edge_index[0].reshape(NW, NPH, M, C)
    dst3 = edge_index[1].reshape(NW, NPH, M, C)
    r = lambda v: v.reshape(1, -1)

    sc_agg = _make_sc_agg()
    (dp,) = _make_sc_deg()(dst3)
    (p1,) = sc_agg(x1, src3, dst3)
    xa = _tc_layer_lrelu(p1, dp, x1, Wl1, Wr1, r(b1), r(g1), r(be1))
    (p2,) = sc_agg(xa, src3, dst3)
    xb = _tc_layer_lrelu(p2, dp, xa, Wl2, Wr2, r(b2), r(g2), r(be2))
    (p3,) = sc_agg(xb, src3, dst3)
    xc = _tc_layer_relu(p3, dp, xb, Wl3, Wr3, r(b3), r(g3), r(be3))
    (p4,) = sc_agg(xc, src3, dst3)
    out = _tc_final(p4, dp, xc, Wl4, Wr4, r(b4), Wm0, r(gm0), r(bm0),
                    Wm1, r(gm1), r(bm1), Wm2, r(bm2))
    return out


# speedup vs baseline: 1.0092x; 1.0092x over previous
import functools

import jax
import jax.numpy as jnp
from jax import lax
from jax.experimental import pallas as pl
from jax.experimental.pallas import tpu as pltpu
from jax.experimental.pallas import tpu_sc as plsc

N = 10000
D = 128
E = 320000
NC, NS = 2, 16
NW = NC * NS
C = 80
PER_W = E // NW
CPW = PER_W // C
NPH = 5
M = CPW // NPH
ZC = 40
NB = 4
GL = 2
SL = 1
NP = 10240
RPT = NP // NS
ZR = 128
DW = 128
BN = 2000
RSCALE = 1.0 / (1.0 + 1e-05) ** 0.5


def _sc_agg_body(dim, c, m, nph, x_hbm, src_hbm, dst_hbm, out_hbm,
                 acc, sidx, didx, rows, *sems):
    gsems, ssems = sems[:NB], sems[NB:]
    cc = lax.axis_index("c")
    ss = lax.axis_index("s")
    wid = cc * NS + ss

    def zrow(r, carry):
        for q in range(dim // 16):
            rows[0, r, pl.ds(q * 16, 16)] = jnp.zeros((16,), jnp.float32)
        return carry
    lax.fori_loop(0, ZC, zrow, None)
    for k in range(RPT // ZC):
        pltpu.sync_copy(rows.at[0, pl.ds(0, ZC)],
                        acc.at[pl.ds(ss * RPT + k * ZC, ZC)])
    plsc.subcore_barrier()

    def fire_g(j, b):
        pltpu.async_copy(x_hbm.at[sidx.at[j]], rows.at[b], gsems[b])

    def wait_g(b):
        pltpu.make_async_copy(x_hbm.at[pl.ds(0, c)], rows.at[b],
                              gsems[b]).wait()

    def fire_s(j, b):
        pltpu.async_copy(rows.at[b], acc.at[didx.at[j]], ssems[b], add=True)

    def wait_s(b):
        pltpu.make_async_copy(rows.at[b], acc.at[pl.ds(0, c)],
                              ssems[b]).wait()

    M = m
    for ph in range(nph):
        pltpu.sync_copy(src_hbm.at[wid, ph], sidx)
        pltpu.sync_copy(dst_hbm.at[wid, ph], didx)
        for j in range(GL):
            fire_g(j, j % NB)
        for j in range(GL, NB):
            fire_g(j, j % NB)
            wait_g((j - GL) % NB)
            fire_s(j - GL, (j - GL) % NB)

        def steady(t, carry):
            for b in range(NB):
                j = NB + t * NB + b
                wait_s(b)
                fire_g(j, b)
                bp = (b - GL) % NB
                wait_g(bp)
                fire_s(j - GL, bp)
            return carry
        nsteady = (M - NB) // NB
        lax.fori_loop(0, nsteady, steady, None)
        for j in range(NB + nsteady * NB, M):
            b = j % NB
            wait_s(b)
            fire_g(j, b)
            bp = (j - GL) % NB
            wait_g(bp)
            fire_s(j - GL, bp)
        for jj in range(M - GL, M):
            bp = jj % NB
            wait_g(bp)
            fire_s(jj, bp)
        for b in range(NB):
            wait_s(b)

    plsc.subcore_barrier()
    pltpu.sync_copy(acc.at[pl.ds(ss * RPT, RPT)],
                    out_hbm.at[cc, pl.ds(ss * RPT, RPT)])


@functools.lru_cache(maxsize=None)
def _make_sc_agg(dim=D, c=C, nph=NPH):
    m = PER_W // c // nph
    mesh = plsc.VectorSubcoreMesh(core_axis_name="c", subcore_axis_name="s",
                                  num_cores=NC, num_subcores=NS)
    return pl.kernel(
        functools.partial(_sc_agg_body, dim, c, m, nph),
        out_type=(jax.ShapeDtypeStruct((NC, NP, dim), jnp.float32),),
        mesh=mesh,
        scratch_types=(
            pltpu.VMEM_SHARED((NP, dim), jnp.float32),
            pltpu.VMEM((m, c), jnp.int32),
            pltpu.VMEM((m, c), jnp.int32),
            pltpu.VMEM((NB, c, dim), jnp.float32),
        ) + (pltpu.SemaphoreType.DMA,) * (2 * NB),
    )


def _sc_deg_body(dst_hbm, dout_hbm, dacc, didx, ones, ssem):
    cc = lax.axis_index("c")
    ss = lax.axis_index("s")
    wid = cc * NS + ss

    def zrow(r, carry):
        for q in range(DW // 16):
            ones[r, pl.ds(q * 16, 16)] = jnp.zeros((16,), jnp.float32)
        return carry
    lax.fori_loop(0, C, zrow, None)
    for k in range(RPT // ZC):
        pltpu.sync_copy(ones.at[pl.ds(0, ZC)],
                        dacc.at[pl.ds(ss * RPT + k * ZC, ZC)])

    def onesrow(r, carry):
        for q in range(DW // 16):
            ones[r, pl.ds(q * 16, 16)] = jnp.ones((16,), jnp.float32)
        return carry
    lax.fori_loop(0, C, onesrow, None)
    plsc.subcore_barrier()

    def wait_s():
        pltpu.make_async_copy(ones, dacc.at[pl.ds(0, C)], ssem).wait()

    for ph in range(NPH):
        pltpu.sync_copy(dst_hbm.at[wid, ph], didx)
        for j in range(SL + GL):
            pltpu.async_copy(ones, dacc.at[didx.at[j]], ssem, add=True)

        def chunk(j, carry):
            pltpu.async_copy(ones, dacc.at[didx.at[j]], ssem, add=True)
            wait_s()
            return carry
        lax.fori_loop(SL + GL, M, chunk, None)
        for _ in range(SL + GL):
            wait_s()

    plsc.subcore_barrier()
    pltpu.sync_copy(dacc.at[pl.ds(ss * RPT, RPT)],
                    dout_hbm.at[cc, pl.ds(ss * RPT, RPT)])


@functools.lru_cache(maxsize=None)
def _make_sc_deg():
    mesh = plsc.VectorSubcoreMesh(core_axis_name="c", subcore_axis_name="s",
                                  num_cores=NC, num_subcores=NS)
    return pl.kernel(
        _sc_deg_body,
        out_type=(jax.ShapeDtypeStruct((NC, NP, DW), jnp.float32),),
        mesh=mesh,
        scratch_types=(
            pltpu.VMEM_SHARED((NP, DW), jnp.float32),
            pltpu.VMEM((M, C), jnp.int32),
            pltpu.VMEM((C, DW), jnp.float32),
            pltpu.SemaphoreType.DMA,
        ),
    )


def _tc_dinv_body(d_ref, o_ref):
    deg = jnp.maximum(d_ref[0, :, 0:1] + d_ref[1, :, 0:1], 1.0)
    o_ref[...] = jnp.broadcast_to(1.0 / deg, (NP // 5, 8))


_tc_dinv = pl.pallas_call(
    _tc_dinv_body,
    grid=(5,),
    in_specs=[pl.BlockSpec((NC, NP // 5, DW), lambda i: (0, i, 0))],
    out_specs=pl.BlockSpec((NP // 5, 8), lambda i: (i, 0)),
    out_shape=jax.ShapeDtypeStruct((NP, 8), jnp.float32),
)


def _tc_layer_body(act, p_ref, d_ref, x_ref, wl_ref, wr_ref, b_ref,
                   g_ref, be_ref, o_ref):
    agg = (p_ref[0] + p_ref[1]) * d_ref[:, 0:1]
    y = (jnp.dot(agg, wl_ref[...], preferred_element_type=jnp.float32)
         + jnp.dot(x_ref[...], wr_ref[...], preferred_element_type=jnp.float32)
         + b_ref[...])
    if act == "lrelu":
        y = jnp.where(y >= 0, y, 0.01 * y)
    else:
        y = jnp.maximum(y, 0.0)
    o_ref[...] = y * (RSCALE * g_ref[...]) + be_ref[...]


def _make_tc_layer(act):
    grid = (N // BN,)
    in_specs = [
        pl.BlockSpec((NC, BN, D), lambda i: (0, i, 0)),
        pl.BlockSpec((BN, 8), lambda i: (i, 0)),
        pl.BlockSpec((BN, D), lambda i: (i, 0)),
        pl.BlockSpec((D, D), lambda i: (0, 0)),
        pl.BlockSpec((D, D), lambda i: (0, 0)),
        pl.BlockSpec((1, D), lambda i: (0, 0)),
        pl.BlockSpec((1, D), lambda i: (0, 0)),
        pl.BlockSpec((1, D), lambda i: (0, 0)),
    ]
    return pl.pallas_call(
        functools.partial(_tc_layer_body, act),
        grid=grid,
        in_specs=in_specs,
        out_specs=pl.BlockSpec((BN, D), lambda i: (i, 0)),
        out_shape=jax.ShapeDtypeStruct((N, D), jnp.float32),
    )


_tc_layer_lrelu = _make_tc_layer("lrelu")
_tc_layer_relu = _make_tc_layer("relu")


def _tc_final_body(p_ref, d_ref, x_ref, wl_ref, wr_ref, b_ref,
                   wm0_ref, gm0_ref, bm0_ref, wm1_ref, gm1_ref, bm1_ref,
                   wm2_ref, bm2_ref, o_ref):
    agg = (p_ref[0] + p_ref[1]) * d_ref[:, 0:1]
    y = (jnp.dot(agg, wl_ref[...], preferred_element_type=jnp.float32)
         + jnp.dot(x_ref[...], wr_ref[...], preferred_element_type=jnp.float32)
         + b_ref[...])
    h = jnp.dot(y, wm0_ref[...], preferred_element_type=jnp.float32)
    h = jnp.maximum(h * (RSCALE * gm0_ref[...]) + bm0_ref[...], 0.0)
    h = jnp.dot(h, wm1_ref[...], preferred_element_type=jnp.float32)
    h = jnp.maximum(h * (RSCALE * gm1_ref[...]) + bm1_ref[...], 0.0)
    o_ref[...] = (jnp.dot(h, wm2_ref[...], preferred_element_type=jnp.float32)
                  + bm2_ref[...])


_NCLS = 21

_tc_final = pl.pallas_call(
    _tc_final_body,
    grid=(N // BN,),
    in_specs=[
        pl.BlockSpec((NC, BN, D), lambda i: (0, i, 0)),
        pl.BlockSpec((BN, 8), lambda i: (i, 0)),
        pl.BlockSpec((BN, D), lambda i: (i, 0)),
        pl.BlockSpec((D, D), lambda i: (0, 0)),
        pl.BlockSpec((D, D), lambda i: (0, 0)),
        pl.BlockSpec((1, D), lambda i: (0, 0)),
        pl.BlockSpec((D, D), lambda i: (0, 0)),
        pl.BlockSpec((1, D), lambda i: (0, 0)),
        pl.BlockSpec((1, D), lambda i: (0, 0)),
        pl.BlockSpec((D, D), lambda i: (0, 0)),
        pl.BlockSpec((1, D), lambda i: (0, 0)),
        pl.BlockSpec((1, D), lambda i: (0, 0)),
        pl.BlockSpec((D, _NCLS), lambda i: (0, 0)),
        pl.BlockSpec((1, _NCLS), lambda i: (0, 0)),
    ],
    out_specs=pl.BlockSpec((BN, _NCLS), lambda i: (i, 0)),
    out_shape=jax.ShapeDtypeStruct((N, _NCLS), jnp.float32),
)


def kernel(x1, edge_index, Wl1, Wr1, b1, Wl2, Wr2, b2, Wl3, Wr3, b3,
           Wl4, Wr4, b4, g1, be1, g2, be2, g3, be3,
           Wm0, gm0, bm0, Wm1, gm1, bm1, Wm2, bm2):
    src3 = edge_index[0].reshape(NW, NPH, M, C)
    dst3 = edge_index[1].reshape(NW, NPH, M, C)
    r = lambda v: v.reshape(1, -1)

    sc_agg = _make_sc_agg()
    (dp,) = _make_sc_deg()(dst3)
    dp = _tc_dinv(dp)
    (p1,) = sc_agg(x1, src3, dst3)
    xa = _tc_layer_lrelu(p1, dp, x1, Wl1, Wr1, r(b1), r(g1), r(be1))
    (p2,) = sc_agg(xa, src3, dst3)
    xb = _tc_layer_lrelu(p2, dp, xa, Wl2, Wr2, r(b2), r(g2), r(be2))
    (p3,) = sc_agg(xb, src3, dst3)
    xc = _tc_layer_relu(p3, dp, xb, Wl3, Wr3, r(b3), r(g3), r(be3))
    (p4,) = sc_agg(xc, src3, dst3)
    out = _tc_final(p4, dp, xc, Wl4, Wr4, r(b4), Wm0, r(gm0), r(bm0),
                    Wm1, r(gm1), r(bm1), Wm2, r(bm2))
    return out


# speedup vs baseline: 1.0386x; 1.0291x over previous
import functools

import jax
import jax.numpy as jnp
from jax import lax
from jax.experimental import pallas as pl
from jax.experimental.pallas import tpu as pltpu
from jax.experimental.pallas import tpu_sc as plsc

N = 10000
D = 128
E = 320000
NC, NS = 2, 16
NW = NC * NS
C = 40
PER_W = E // NW
CPW = PER_W // C
NPH = 5
M = CPW // NPH
ZC = 40
NB = 5
GL = 3
SL = 1
NP = 10240
RPT = NP // NS
ZR = 128
DW = 128
BN = 2000
RSCALE = 1.0 / (1.0 + 1e-05) ** 0.5


def _sc_agg_body(dim, c, m, nph, x_hbm, src_hbm, dst_hbm, out_hbm,
                 acc, sidx, didx, rows, *sems):
    gsems, ssems = sems[:NB], sems[NB:]
    cc = lax.axis_index("c")
    ss = lax.axis_index("s")
    wid = cc * NS + ss

    def zrow(r, carry):
        for q in range(dim // 16):
            rows[0, r, pl.ds(q * 16, 16)] = jnp.zeros((16,), jnp.float32)
        return carry
    lax.fori_loop(0, ZC, zrow, None)
    for k in range(RPT // ZC):
        pltpu.sync_copy(rows.at[0, pl.ds(0, ZC)],
                        acc.at[pl.ds(ss * RPT + k * ZC, ZC)])
    plsc.subcore_barrier()

    def fire_g(j, b):
        pltpu.async_copy(x_hbm.at[sidx.at[j]], rows.at[b], gsems[b])

    def wait_g(b):
        pltpu.make_async_copy(x_hbm.at[pl.ds(0, c)], rows.at[b],
                              gsems[b]).wait()

    def fire_s(j, b):
        pltpu.async_copy(rows.at[b], acc.at[didx.at[j]], ssems[b], add=True)

    def wait_s(b):
        pltpu.make_async_copy(rows.at[b], acc.at[pl.ds(0, c)],
                              ssems[b]).wait()

    M = m
    for ph in range(nph):
        pltpu.sync_copy(src_hbm.at[wid, ph], sidx)
        pltpu.sync_copy(dst_hbm.at[wid, ph], didx)
        for j in range(GL):
            fire_g(j, j % NB)
        for j in range(GL, NB):
            fire_g(j, j % NB)
            wait_g((j - GL) % NB)
            fire_s(j - GL, (j - GL) % NB)

        def steady(t, carry):
            for b in range(NB):
                j = NB + t * NB + b
                wait_s(b)
                fire_g(j, b)
                bp = (b - GL) % NB
                wait_g(bp)
                fire_s(j - GL, bp)
            return carry
        nsteady = (M - NB) // NB
        lax.fori_loop(0, nsteady, steady, None)
        for j in range(NB + nsteady * NB, M):
            b = j % NB
            wait_s(b)
            fire_g(j, b)
            bp = (j - GL) % NB
            wait_g(bp)
            fire_s(j - GL, bp)
        for jj in range(M - GL, M):
            bp = jj % NB
            wait_g(bp)
            fire_s(jj, bp)
        for b in range(NB):
            wait_s(b)

    plsc.subcore_barrier()
    pltpu.sync_copy(acc.at[pl.ds(ss * RPT, RPT)],
                    out_hbm.at[cc, pl.ds(ss * RPT, RPT)])


@functools.lru_cache(maxsize=None)
def _make_sc_agg(dim=D, c=C, nph=NPH):
    m = PER_W // c // nph
    mesh = plsc.VectorSubcoreMesh(core_axis_name="c", subcore_axis_name="s",
                                  num_cores=NC, num_subcores=NS)
    return pl.kernel(
        functools.partial(_sc_agg_body, dim, c, m, nph),
        out_type=(jax.ShapeDtypeStruct((NC, NP, dim), jnp.float32),),
        mesh=mesh,
        scratch_types=(
            pltpu.VMEM_SHARED((NP, dim), jnp.float32),
            pltpu.VMEM((m, c), jnp.int32),
            pltpu.VMEM((m, c), jnp.int32),
            pltpu.VMEM((NB, c, dim), jnp.float32),
        ) + (pltpu.SemaphoreType.DMA,) * (2 * NB),
    )


def _sc_deg_body(dst_hbm, dout_hbm, dacc, didx, ones, ssem):
    cc = lax.axis_index("c")
    ss = lax.axis_index("s")
    wid = cc * NS + ss

    def zrow(r, carry):
        for q in range(DW // 16):
            ones[r, pl.ds(q * 16, 16)] = jnp.zeros((16,), jnp.float32)
        return carry
    lax.fori_loop(0, C, zrow, None)
    for k in range(RPT // ZC):
        pltpu.sync_copy(ones.at[pl.ds(0, ZC)],
                        dacc.at[pl.ds(ss * RPT + k * ZC, ZC)])

    def onesrow(r, carry):
        for q in range(DW // 16):
            ones[r, pl.ds(q * 16, 16)] = jnp.ones((16,), jnp.float32)
        return carry
    lax.fori_loop(0, C, onesrow, None)
    plsc.subcore_barrier()

    def wait_s():
        pltpu.make_async_copy(ones, dacc.at[pl.ds(0, C)], ssem).wait()

    for ph in range(NPH):
        pltpu.sync_copy(dst_hbm.at[wid, ph], didx)
        for j in range(SL + GL):
            pltpu.async_copy(ones, dacc.at[didx.at[j]], ssem, add=True)

        def chunk(j, carry):
            pltpu.async_copy(ones, dacc.at[didx.at[j]], ssem, add=True)
            wait_s()
            return carry
        lax.fori_loop(SL + GL, M, chunk, None)
        for _ in range(SL + GL):
            wait_s()

    plsc.subcore_barrier()
    pltpu.sync_copy(dacc.at[pl.ds(ss * RPT, RPT)],
                    dout_hbm.at[cc, pl.ds(ss * RPT, RPT)])


@functools.lru_cache(maxsize=None)
def _make_sc_deg():
    mesh = plsc.VectorSubcoreMesh(core_axis_name="c", subcore_axis_name="s",
                                  num_cores=NC, num_subcores=NS)
    return pl.kernel(
        _sc_deg_body,
        out_type=(jax.ShapeDtypeStruct((NC, NP, DW), jnp.float32),),
        mesh=mesh,
        scratch_types=(
            pltpu.VMEM_SHARED((NP, DW), jnp.float32),
            pltpu.VMEM((M, C), jnp.int32),
            pltpu.VMEM((C, DW), jnp.float32),
            pltpu.SemaphoreType.DMA,
        ),
    )


def _tc_dinv_body(d_ref, o_ref):
    deg = jnp.maximum(d_ref[0, :, 0:1] + d_ref[1, :, 0:1], 1.0)
    o_ref[...] = jnp.broadcast_to(1.0 / deg, (NP // 5, 8))


_tc_dinv = pl.pallas_call(
    _tc_dinv_body,
    grid=(5,),
    in_specs=[pl.BlockSpec((NC, NP // 5, DW), lambda i: (0, i, 0))],
    out_specs=pl.BlockSpec((NP // 5, 8), lambda i: (i, 0)),
    out_shape=jax.ShapeDtypeStruct((NP, 8), jnp.float32),
)


def _tc_layer_body(act, p_ref, d_ref, x_ref, wl_ref, wr_ref, b_ref,
                   g_ref, be_ref, o_ref):
    agg = (p_ref[0] + p_ref[1]) * d_ref[:, 0:1]
    y = (jnp.dot(agg, wl_ref[...], preferred_element_type=jnp.float32)
         + jnp.dot(x_ref[...], wr_ref[...], preferred_element_type=jnp.float32)
         + b_ref[...])
    if act == "lrelu":
        y = jnp.where(y >= 0, y, 0.01 * y)
    else:
        y = jnp.maximum(y, 0.0)
    o_ref[...] = y * (RSCALE * g_ref[...]) + be_ref[...]


def _make_tc_layer(act):
    grid = (N // BN,)
    in_specs = [
        pl.BlockSpec((NC, BN, D), lambda i: (0, i, 0)),
        pl.BlockSpec((BN, 8), lambda i: (i, 0)),
        pl.BlockSpec((BN, D), lambda i: (i, 0)),
        pl.BlockSpec((D, D), lambda i: (0, 0)),
        pl.BlockSpec((D, D), lambda i: (0, 0)),
        pl.BlockSpec((1, D), lambda i: (0, 0)),
        pl.BlockSpec((1, D), lambda i: (0, 0)),
        pl.BlockSpec((1, D), lambda i: (0, 0)),
    ]
    return pl.pallas_call(
        functools.partial(_tc_layer_body, act),
        grid=grid,
        in_specs=in_specs,
        out_specs=pl.BlockSpec((BN, D), lambda i: (i, 0)),
        out_shape=jax.ShapeDtypeStruct((N, D), jnp.float32),
    )


_tc_layer_lrelu = _make_tc_layer("lrelu")
_tc_layer_relu = _make_tc_layer("relu")


def _tc_final_body(p_ref, d_ref, x_ref, wl_ref, wr_ref, b_ref,
                   wm0_ref, gm0_ref, bm0_ref, wm1_ref, gm1_ref, bm1_ref,
                   wm2_ref, bm2_ref, o_ref):
    agg = (p_ref[0] + p_ref[1]) * d_ref[:, 0:1]
    y = (jnp.dot(agg, wl_ref[...], preferred_element_type=jnp.float32)
         + jnp.dot(x_ref[...], wr_ref[...], preferred_element_type=jnp.float32)
         + b_ref[...])
    h = jnp.dot(y, wm0_ref[...], preferred_element_type=jnp.float32)
    h = jnp.maximum(h * (RSCALE * gm0_ref[...]) + bm0_ref[...], 0.0)
    h = jnp.dot(h, wm1_ref[...], preferred_element_type=jnp.float32)
    h = jnp.maximum(h * (RSCALE * gm1_ref[...]) + bm1_ref[...], 0.0)
    o_ref[...] = (jnp.dot(h, wm2_ref[...], preferred_element_type=jnp.float32)
                  + bm2_ref[...])


_NCLS = 21

_tc_final = pl.pallas_call(
    _tc_final_body,
    grid=(N // BN,),
    in_specs=[
        pl.BlockSpec((NC, BN, D), lambda i: (0, i, 0)),
        pl.BlockSpec((BN, 8), lambda i: (i, 0)),
        pl.BlockSpec((BN, D), lambda i: (i, 0)),
        pl.BlockSpec((D, D), lambda i: (0, 0)),
        pl.BlockSpec((D, D), lambda i: (0, 0)),
        pl.BlockSpec((1, D), lambda i: (0, 0)),
        pl.BlockSpec((D, D), lambda i: (0, 0)),
        pl.BlockSpec((1, D), lambda i: (0, 0)),
        pl.BlockSpec((1, D), lambda i: (0, 0)),
        pl.BlockSpec((D, D), lambda i: (0, 0)),
        pl.BlockSpec((1, D), lambda i: (0, 0)),
        pl.BlockSpec((1, D), lambda i: (0, 0)),
        pl.BlockSpec((D, _NCLS), lambda i: (0, 0)),
        pl.BlockSpec((1, _NCLS), lambda i: (0, 0)),
    ],
    out_specs=pl.BlockSpec((BN, _NCLS), lambda i: (i, 0)),
    out_shape=jax.ShapeDtypeStruct((N, _NCLS), jnp.float32),
)


def kernel(x1, edge_index, Wl1, Wr1, b1, Wl2, Wr2, b2, Wl3, Wr3, b3,
           Wl4, Wr4, b4, g1, be1, g2, be2, g3, be3,
           Wm0, gm0, bm0, Wm1, gm1, bm1, Wm2, bm2):
    src3 = edge_index[0].reshape(NW, NPH, M, C)
    dst3 = edge_index[1].reshape(NW, NPH, M, C)
    r = lambda v: v.reshape(1, -1)

    sc_agg = _make_sc_agg()
    (dp,) = _make_sc_deg()(dst3)
    dp = _tc_dinv(dp)
    (p1,) = sc_agg(x1, src3, dst3)
    xa = _tc_layer_lrelu(p1, dp, x1, Wl1, Wr1, r(b1), r(g1), r(be1))
    (p2,) = sc_agg(xa, src3, dst3)
    xb = _tc_layer_lrelu(p2, dp, xa, Wl2, Wr2, r(b2), r(g2), r(be2))
    (p3,) = sc_agg(xb, src3, dst3)
    xc = _tc_layer_relu(p3, dp, xb, Wl3, Wr3, r(b3), r(g3), r(be3))
    (p4,) = sc_agg(xc, src3, dst3)
    out = _tc_final(p4, dp, xc, Wl4, Wr4, r(b4), Wm0, r(gm0), r(bm0),
                    Wm1, r(gm1), r(bm1), Wm2, r(bm2))
    return out


# speedup vs baseline: 1.0566x; 1.0174x over previous
import functools

import jax
import jax.numpy as jnp
from jax import lax
from jax.experimental import pallas as pl
from jax.experimental.pallas import tpu as pltpu
from jax.experimental.pallas import tpu_sc as plsc

N = 10000
D = 128
E = 320000
NC, NS = 2, 16
NW = NC * NS
C = 40
PER_W = E // NW
CPW = PER_W // C
NPH = 5
M = CPW // NPH
ZC = 40
NB = 6
GL = 4
SL = 1
NP = 10240
RPT = NP // NS
ZR = 128
DW = 128
BN = 2000
RSCALE = 1.0 / (1.0 + 1e-05) ** 0.5


def _sc_agg_body(dim, c, m, nph, x_hbm, src_hbm, dst_hbm, out_hbm,
                 acc, sidx, didx, rows, *sems):
    gsems, ssems = sems[:NB], sems[NB:]
    cc = lax.axis_index("c")
    ss = lax.axis_index("s")
    wid = cc * NS + ss

    def zrow(r, carry):
        for q in range(dim // 16):
            rows[0, r, pl.ds(q * 16, 16)] = jnp.zeros((16,), jnp.float32)
        return carry
    lax.fori_loop(0, ZC, zrow, None)
    for k in range(RPT // ZC):
        pltpu.sync_copy(rows.at[0, pl.ds(0, ZC)],
                        acc.at[pl.ds(ss * RPT + k * ZC, ZC)])
    plsc.subcore_barrier()

    def fire_g(j, b):
        pltpu.async_copy(x_hbm.at[sidx.at[j]], rows.at[b], gsems[b])

    def wait_g(b):
        pltpu.make_async_copy(x_hbm.at[pl.ds(0, c)], rows.at[b],
                              gsems[b]).wait()

    def fire_s(j, b):
        pltpu.async_copy(rows.at[b], acc.at[didx.at[j]], ssems[b], add=True)

    def wait_s(b):
        pltpu.make_async_copy(rows.at[b], acc.at[pl.ds(0, c)],
                              ssems[b]).wait()

    M = m
    for ph in range(nph):
        pltpu.sync_copy(src_hbm.at[wid, ph], sidx)
        pltpu.sync_copy(dst_hbm.at[wid, ph], didx)
        for j in range(GL):
            fire_g(j, j % NB)
        for j in range(GL, NB):
            fire_g(j, j % NB)
            wait_g((j - GL) % NB)
            fire_s(j - GL, (j - GL) % NB)

        def steady(t, carry):
            for b in range(NB):
                j = NB + t * NB + b
                wait_s(b)
                fire_g(j, b)
                bp = (b - GL) % NB
                wait_g(bp)
                fire_s(j - GL, bp)
            return carry
        nsteady = (M - NB) // NB
        lax.fori_loop(0, nsteady, steady, None)
        for j in range(NB + nsteady * NB, M):
            b = j % NB
            wait_s(b)
            fire_g(j, b)
            bp = (j - GL) % NB
            wait_g(bp)
            fire_s(j - GL, bp)
        for jj in range(M - GL, M):
            bp = jj % NB
            wait_g(bp)
            fire_s(jj, bp)
        for b in range(NB):
            wait_s(b)

    plsc.subcore_barrier()
    pltpu.sync_copy(acc.at[pl.ds(ss * RPT, RPT)],
                    out_hbm.at[cc, pl.ds(ss * RPT, RPT)])


@functools.lru_cache(maxsize=None)
def _make_sc_agg(dim=D, c=C, nph=NPH):
    m = PER_W // c // nph
    mesh = plsc.VectorSubcoreMesh(core_axis_name="c", subcore_axis_name="s",
                                  num_cores=NC, num_subcores=NS)
    return pl.kernel(
        functools.partial(_sc_agg_body, dim, c, m, nph),
        out_type=(jax.ShapeDtypeStruct((NC, NP, dim), jnp.float32),),
        mesh=mesh,
        scratch_types=(
            pltpu.VMEM_SHARED((NP, dim), jnp.float32),
            pltpu.VMEM((m, c), jnp.int32),
            pltpu.VMEM((m, c), jnp.int32),
            pltpu.VMEM((NB, c, dim), jnp.float32),
        ) + (pltpu.SemaphoreType.DMA,) * (2 * NB),
    )


def _sc_deg_body(dst_hbm, dout_hbm, dacc, didx, ones, ssem):
    cc = lax.axis_index("c")
    ss = lax.axis_index("s")
    wid = cc * NS + ss

    def zrow(r, carry):
        for q in range(DW // 16):
            ones[r, pl.ds(q * 16, 16)] = jnp.zeros((16,), jnp.float32)
        return carry
    lax.fori_loop(0, C, zrow, None)
    for k in range(RPT // ZC):
        pltpu.sync_copy(ones.at[pl.ds(0, ZC)],
                        dacc.at[pl.ds(ss * RPT + k * ZC, ZC)])

    def onesrow(r, carry):
        for q in range(DW // 16):
            ones[r, pl.ds(q * 16, 16)] = jnp.ones((16,), jnp.float32)
        return carry
    lax.fori_loop(0, C, onesrow, None)
    plsc.subcore_barrier()

    def wait_s():
        pltpu.make_async_copy(ones, dacc.at[pl.ds(0, C)], ssem).wait()

    for ph in range(NPH):
        pltpu.sync_copy(dst_hbm.at[wid, ph], didx)
        for j in range(SL + GL):
            pltpu.async_copy(ones, dacc.at[didx.at[j]], ssem, add=True)

        def chunk(j, carry):
            pltpu.async_copy(ones, dacc.at[didx.at[j]], ssem, add=True)
            wait_s()
            return carry
        lax.fori_loop(SL + GL, M, chunk, None)
        for _ in range(SL + GL):
            wait_s()

    plsc.subcore_barrier()
    pltpu.sync_copy(dacc.at[pl.ds(ss * RPT, RPT)],
                    dout_hbm.at[cc, pl.ds(ss * RPT, RPT)])


@functools.lru_cache(maxsize=None)
def _make_sc_deg():
    mesh = plsc.VectorSubcoreMesh(core_axis_name="c", subcore_axis_name="s",
                                  num_cores=NC, num_subcores=NS)
    return pl.kernel(
        _sc_deg_body,
        out_type=(jax.ShapeDtypeStruct((NC, NP, DW), jnp.float32),),
        mesh=mesh,
        scratch_types=(
            pltpu.VMEM_SHARED((NP, DW), jnp.float32),
            pltpu.VMEM((M, C), jnp.int32),
            pltpu.VMEM((C, DW), jnp.float32),
            pltpu.SemaphoreType.DMA,
        ),
    )


def _tc_dinv_body(d_ref, o_ref):
    deg = jnp.maximum(d_ref[0, :, 0:1] + d_ref[1, :, 0:1], 1.0)
    o_ref[...] = jnp.broadcast_to(1.0 / deg, (NP // 5, 8))


_tc_dinv = pl.pallas_call(
    _tc_dinv_body,
    grid=(5,),
    in_specs=[pl.BlockSpec((NC, NP // 5, DW), lambda i: (0, i, 0))],
    out_specs=pl.BlockSpec((NP // 5, 8), lambda i: (i, 0)),
    out_shape=jax.ShapeDtypeStruct((NP, 8), jnp.float32),
)


def _tc_layer_body(act, p_ref, d_ref, x_ref, wl_ref, wr_ref, b_ref,
                   g_ref, be_ref, o_ref):
    agg = (p_ref[0] + p_ref[1]) * d_ref[:, 0:1]
    y = (jnp.dot(agg, wl_ref[...], preferred_element_type=jnp.float32)
         + jnp.dot(x_ref[...], wr_ref[...], preferred_element_type=jnp.float32)
         + b_ref[...])
    if act == "lrelu":
        y = jnp.where(y >= 0, y, 0.01 * y)
    else:
        y = jnp.maximum(y, 0.0)
    o_ref[...] = y * (RSCALE * g_ref[...]) + be_ref[...]


def _make_tc_layer(act):
    grid = (N // BN,)
    in_specs = [
        pl.BlockSpec((NC, BN, D), lambda i: (0, i, 0)),
        pl.BlockSpec((BN, 8), lambda i: (i, 0)),
        pl.BlockSpec((BN, D), lambda i: (i, 0)),
        pl.BlockSpec((D, D), lambda i: (0, 0)),
        pl.BlockSpec((D, D), lambda i: (0, 0)),
        pl.BlockSpec((1, D), lambda i: (0, 0)),
        pl.BlockSpec((1, D), lambda i: (0, 0)),
        pl.BlockSpec((1, D), lambda i: (0, 0)),
    ]
    return pl.pallas_call(
        functools.partial(_tc_layer_body, act),
        grid=grid,
        in_specs=in_specs,
        out_specs=pl.BlockSpec((BN, D), lambda i: (i, 0)),
        out_shape=jax.ShapeDtypeStruct((N, D), jnp.float32),
    )


_tc_layer_lrelu = _make_tc_layer("lrelu")
_tc_layer_relu = _make_tc_layer("relu")


def _tc_final_body(p_ref, d_ref, x_ref, wl_ref, wr_ref, b_ref,
                   wm0_ref, gm0_ref, bm0_ref, wm1_ref, gm1_ref, bm1_ref,
                   wm2_ref, bm2_ref, o_ref):
    agg = (p_ref[0] + p_ref[1]) * d_ref[:, 0:1]
    y = (jnp.dot(agg, wl_ref[...], preferred_element_type=jnp.float32)
         + jnp.dot(x_ref[...], wr_ref[...], preferred_element_type=jnp.float32)
         + b_ref[...])
    h = jnp.dot(y, wm0_ref[...], preferred_element_type=jnp.float32)
    h = jnp.maximum(h * (RSCALE * gm0_ref[...]) + bm0_ref[...], 0.0)
    h = jnp.dot(h, wm1_ref[...], preferred_element_type=jnp.float32)
    h = jnp.maximum(h * (RSCALE * gm1_ref[...]) + bm1_ref[...], 0.0)
    o_ref[...] = (jnp.dot(h, wm2_ref[...], preferred_element_type=jnp.float32)
                  + bm2_ref[...])


_NCLS = 21

_tc_final = pl.pallas_call(
    _tc_final_body,
    grid=(N // BN,),
    in_specs=[
        pl.BlockSpec((NC, BN, D), lambda i: (0, i, 0)),
        pl.BlockSpec((BN, 8), lambda i: (i, 0)),
        pl.BlockSpec((BN, D), lambda i: (i, 0)),
        pl.BlockSpec((D, D), lambda i: (0, 0)),
        pl.BlockSpec((D, D), lambda i: (0, 0)),
        pl.BlockSpec((1, D), lambda i: (0, 0)),
        pl.BlockSpec((D, D), lambda i: (0, 0)),
        pl.BlockSpec((1, D), lambda i: (0, 0)),
        pl.BlockSpec((1, D), lambda i: (0, 0)),
        pl.BlockSpec((D, D), lambda i: (0, 0)),
        pl.BlockSpec((1, D), lambda i: (0, 0)),
        pl.BlockSpec((1, D), lambda i: (0, 0)),
        pl.BlockSpec((D, _NCLS), lambda i: (0, 0)),
        pl.BlockSpec((1, _NCLS), lambda i: (0, 0)),
    ],
    out_specs=pl.BlockSpec((BN, _NCLS), lambda i: (i, 0)),
    out_shape=jax.ShapeDtypeStruct((N, _NCLS), jnp.float32),
)


def kernel(x1, edge_index, Wl1, Wr1, b1, Wl2, Wr2, b2, Wl3, Wr3, b3,
           Wl4, Wr4, b4, g1, be1, g2, be2, g3, be3,
           Wm0, gm0, bm0, Wm1, gm1, bm1, Wm2, bm2):
    src3 = edge_index[0].reshape(NW, NPH, M, C)
    dst3 = edge_index[1].reshape(NW, NPH, M, C)
    r = lambda v: v.reshape(1, -1)

    sc_agg = _make_sc_agg()
    (dp,) = _make_sc_deg()(dst3)
    dp = _tc_dinv(dp)
    (p1,) = sc_agg(x1, src3, dst3)
    xa = _tc_layer_lrelu(p1, dp, x1, Wl1, Wr1, r(b1), r(g1), r(be1))
    (p2,) = sc_agg(xa, src3, dst3)
    xb = _tc_layer_lrelu(p2, dp, xa, Wl2, Wr2, r(b2), r(g2), r(be2))
    (p3,) = sc_agg(xb, src3, dst3)
    xc = _tc_layer_relu(p3, dp, xb, Wl3, Wr3, r(b3), r(g3), r(be3))
    (p4,) = sc_agg(xc, src3, dst3)
    out = _tc_final(p4, dp, xc, Wl4, Wr4, r(b4), Wm0, r(gm0), r(bm0),
                    Wm1, r(gm1), r(bm1), Wm2, r(bm2))
    return out
